# baseline (speedup 1.0000x reference)
import functools

import jax
import jax.numpy as jnp
from jax import lax
from jax.experimental import pallas as pl
from jax.experimental.pallas import tpu as pltpu
from jax.experimental.pallas import tpu_sc as plsc

VOCAB = 8192
NUM_ROWS = 4096
NUM_CORES = 2
NUM_SUBCORES = 16
NW = NUM_CORES * NUM_SUBCORES
ROWS_PER_W = NUM_ROWS // NW
K = 4
NBUF = 3
LAG = 1
NCHUNK = ROWS_PER_W // K

_mesh = plsc.VectorSubcoreMesh(core_axis_name="c", subcore_axis_name="s")


@functools.partial(
    pl.kernel,
    mesh=_mesh,
    out_type=jax.ShapeDtypeStruct((NUM_ROWS, VOCAB), jnp.float32),
    scratch_types=[
        pltpu.VMEM((NCHUNK, K), jnp.int32),
        pltpu.VMEM((NBUF, K, VOCAB), jnp.float32),
        pltpu.SemaphoreType.DMA,
        pltpu.SemaphoreType.DMA,
        pltpu.SemaphoreType.DMA,
        pltpu.SemaphoreType.DMA,
        pltpu.SemaphoreType.DMA,
        pltpu.SemaphoreType.DMA,
    ],
)
def _sc_gather(idx_hbm, table_hbm, out_hbm, idx_v, buf, *sems):
    wid = lax.axis_index("s") * NUM_CORES + lax.axis_index("c")
    base = wid * ROWS_PER_W
    sem_in = sems[:NBUF]
    sem_out = sems[NBUF:]
    pltpu.sync_copy(idx_hbm.at[wid], idx_v)

    for s in range(NCHUNK + LAG):
        if s < NCHUNK:
            b = s % NBUF
            if s >= NBUF:
                pltpu.make_async_copy(
                    buf.at[b], out_hbm.at[pl.ds(base + (s - NBUF) * K, K)],
                    sem_out[b],
                ).wait()
            pltpu.async_copy(table_hbm.at[idx_v.at[s]], buf.at[b], sem_in[b])
        if s >= LAG:
            c = s - LAG
            bc = c % NBUF
            pltpu.make_async_copy(
                table_hbm.at[idx_v.at[c]], buf.at[bc], sem_in[bc]
            ).wait()
            pltpu.async_copy(
                buf.at[bc], out_hbm.at[pl.ds(base + c * K, K)], sem_out[bc]
            )

    for c in range(NCHUNK - NBUF, NCHUNK):
        bc = c % NBUF
        pltpu.make_async_copy(
            buf.at[bc], out_hbm.at[pl.ds(base + c * K, K)], sem_out[bc]
        ).wait()


def kernel(idx, table):
    idx_r = idx.reshape(NW, NCHUNK, K).astype(jnp.int32)
    return _sc_gather(idx_r, table)


# speedup vs baseline: 1.0366x; 1.0366x over previous
import functools

import jax
import jax.numpy as jnp
from jax import lax
from jax.experimental import pallas as pl
from jax.experimental.pallas import tpu as pltpu
from jax.experimental.pallas import tpu_sc as plsc

VOCAB = 8192
NUM_ROWS = 4096
NUM_CORES = 2
NUM_SUBCORES = 16
NW = NUM_CORES * NUM_SUBCORES
ROWS_PER_W = NUM_ROWS // NW
K = 4
NBUF = 2
NCHUNK = ROWS_PER_W // K

_mesh = plsc.VectorSubcoreMesh(core_axis_name="c", subcore_axis_name="s")


@functools.partial(
    pl.kernel,
    mesh=_mesh,
    out_type=jax.ShapeDtypeStruct((NUM_ROWS, VOCAB), jnp.float32),
    scratch_types=[
        pltpu.VMEM((NCHUNK, K), jnp.int32),
        pltpu.VMEM((NBUF, K, VOCAB), jnp.float32),
        pltpu.SemaphoreType.DMA,
        pltpu.SemaphoreType.DMA,
    ],
)
def _sc_gather(idx_hbm, table_hbm, out_hbm, idx_v, buf, sem0, sem1):
    wid = lax.axis_index("s") * NUM_CORES + lax.axis_index("c")
    base = wid * ROWS_PER_W
    sems = (sem0, sem1)
    pltpu.sync_copy(idx_hbm.at[wid], idx_v)

    for b in range(NBUF):
        pltpu.async_copy(table_hbm.at[idx_v.at[b]], buf.at[b], sems[b])

    def body(p, carry):
        g = p * NBUF
        for b in range(NBUF):
            cur = g + b
            pltpu.make_async_copy(
                table_hbm.at[idx_v.at[cur]], buf.at[b], sems[b]
            ).wait()
            pltpu.sync_copy(buf.at[b], out_hbm.at[pl.ds(base + cur * K, K)])
            nxt = cur + NBUF

            @pl.when(nxt < NCHUNK)
            def _():
                pltpu.async_copy(table_hbm.at[idx_v.at[nxt]], buf.at[b], sems[b])

        return carry

    lax.fori_loop(0, NCHUNK // NBUF, body, 0)


def kernel(idx, table):
    idx_r = idx.reshape(NW, NCHUNK, K).astype(jnp.int32)
    return _sc_gather(idx_r, table)
